# baseline (speedup 1.0000x reference)
import functools

import jax
import jax.numpy as jnp
from jax import lax
from jax.experimental import pallas as pl
from jax.experimental.pallas import tpu as pltpu
from jax.experimental.pallas import tpu_sc as plsc

D = 64
DP = 128
NW = 32
CHUNK = 128
STREAMS_PER_BUF = 4
BUF_ROWS = CHUNK * STREAMS_PER_BUF

TC = 256
NCH = 999936 // TC


def _repack_call(V):
    mesh = plsc.VectorSubcoreMesh(core_axis_name="c", subcore_axis_name="s")

    @functools.partial(
        pl.kernel,
        mesh=mesh,
        out_type=jax.ShapeDtypeStruct((V // 2, DP), jnp.float32),
        compiler_params=pltpu.CompilerParams(
            use_tc_tiling_on_sc=True, needs_layout_passes=False
        ),
        scratch_types=[
            pltpu.VMEM((D, TC), jnp.float32),
            pltpu.VMEM((D, TC), jnp.float32),
            pltpu.VMEM((TC // 2, DP), jnp.float32),
            pltpu.SemaphoreType.DMA,
            pltpu.SemaphoreType.DMA,
        ],
    )
    def repack(wt_hbm, tail_hbm, out_hbm, in0, in1, outv, s0, s1):
        wid = lax.axis_index("s") * 2 + lax.axis_index("c")
        ins = (in0, in1)
        sems = (s0, s1)
        d_idx = [jax.lax.iota(jnp.int32, 16) + k * 16 for k in range(4)]

        def fire(c, buf, sem):
            pltpu.make_async_copy(
                wt_hbm.at[:, pl.ds(c * TC, TC)], buf, sem
            ).start()

        def drain(buf, sem):
            pltpu.make_async_copy(
                wt_hbm.at[:, pl.ds(0, TC)], buf, sem
            ).wait()

        fire(wid, in0, s0)

        @pl.when(wid + 32 < NCH)
        def _():
            fire(wid + 32, in1, s1)

        def body(p, carry):
            for b in range(2):
                i = 2 * p + b
                c = wid + i * 32

                @pl.when(c < NCH)
                def _():
                    drain(ins[b], sems[b])

                    def ubody(u, carry2):
                        t0 = jnp.full((16,), 2 * u, jnp.int32)
                        t1 = t0 + 1
                        for k in range(4):
                            outv[u, pl.ds(k * 16, 16)] = plsc.load_gather(
                                ins[b], [d_idx[k], t0]
                            )
                            outv[u, pl.ds(64 + k * 16, 16)] = plsc.load_gather(
                                ins[b], [d_idx[k], t1]
                            )
                        return carry2

                    lax.fori_loop(0, TC // 2, ubody, 0)
                    pltpu.sync_copy(
                        outv, out_hbm.at[pl.ds(c * (TC // 2), TC // 2)]
                    )
                    c2 = c + 64

                    @pl.when(c2 < NCH)
                    def _():
                        fire(c2, ins[b], sems[b])

            return carry

        n_iter = (NCH + 31) // 32
        lax.fori_loop(0, (n_iter + 1) // 2, body, 0)

        @pl.when(wid == 0)
        def _():
            pltpu.sync_copy(tail_hbm, outv.at[pl.ds(0, 32)])
            pltpu.sync_copy(
                outv.at[pl.ds(0, 32)], out_hbm.at[pl.ds(V // 2 - 32, 32)]
            )

    return repack


def _emb_call(total):
    b_per_w = total // NW
    n_rows = b_per_w // CHUNK
    n_bufs = b_per_w // BUF_ROWS

    mesh = plsc.VectorSubcoreMesh(core_axis_name="c", subcore_axis_name="s")

    @functools.partial(
        pl.kernel,
        mesh=mesh,
        out_type=jax.ShapeDtypeStruct((total, DP), jnp.float32),
        compiler_params=pltpu.CompilerParams(use_tc_tiling_on_sc=False),
        scratch_types=[
            pltpu.VMEM((n_rows, CHUNK), jnp.int32),
            pltpu.VMEM((BUF_ROWS, D), jnp.float32),
            pltpu.VMEM((BUF_ROWS, D), jnp.float32),
            pltpu.SemaphoreType.DMA,
            pltpu.SemaphoreType.DMA,
        ],
    )
    def emb(idx_hbm, table_hbm, out_hbm, idx_v, rows0, rows1, g0, g1):
        wid = lax.axis_index("s") * 2 + lax.axis_index("c")
        base = wid * b_per_w
        pltpu.sync_copy(idx_hbm.at[wid], idx_v)

        rows = (rows0, rows1)
        gsem = (g0, g1)

        def fire(g, rows_ref, sem):
            for j in range(STREAMS_PER_BUF):
                pltpu.make_async_copy(
                    table_hbm.at[idx_v.at[g * STREAMS_PER_BUF + j]],
                    rows_ref.at[pl.ds(j * CHUNK, CHUNK)],
                    sem,
                ).start()

        def drain(rows_ref, sem):
            pltpu.make_async_copy(
                table_hbm.at[pl.ds(0, BUF_ROWS)], rows_ref, sem
            ).wait()

        fire(0, rows0, g0)

        def body(p, carry):
            for b in range(2):
                g = p * 2 + b
                drain(rows[b], gsem[b])
                if b == 0:
                    fire(g + 1, rows[1], gsem[1])
                else:
                    @pl.when(g + 1 < n_bufs)
                    def _():
                        fire(g + 1, rows[0], gsem[0])
                pltpu.sync_copy(
                    rows[b],
                    out_hbm.at[pl.ds(base + g * BUF_ROWS, BUF_ROWS), pl.ds(0, D)],
                )
            return carry

        lax.fori_loop(0, n_bufs // 2, body, 0)

    return emb


def kernel(token_ids, weight):
    B, S = token_ids.shape
    total = B * S
    V = weight.shape[0]
    idx = token_ids.reshape(NW, total // (NW * CHUNK), CHUNK).astype(jnp.int32)
    wt = weight.T
    tail = weight[NCH * TC:].reshape(V // 2 - NCH * TC // 2, DP)
    table = _repack_call(V)(wt, tail).reshape(V, D)
    out = _emb_call(total)(idx, table)
    return out[:, :D].reshape(B, S, D)


# speedup vs baseline: 1.0025x; 1.0025x over previous
import functools

import jax
import jax.numpy as jnp
from jax import lax
from jax.experimental import pallas as pl
from jax.experimental.pallas import tpu as pltpu
from jax.experimental.pallas import tpu_sc as plsc

D = 64
DP = 128
NW = 32
CHUNK = 128
STREAMS_PER_BUF = 4
BUF_ROWS = CHUNK * STREAMS_PER_BUF

TC = 256
NCH = 999936 // TC


def _repack_call(V):
    mesh = plsc.VectorSubcoreMesh(core_axis_name="c", subcore_axis_name="s")

    @functools.partial(
        pl.kernel,
        mesh=mesh,
        out_type=jax.ShapeDtypeStruct((V // 2, DP), jnp.float32),
        compiler_params=pltpu.CompilerParams(
            use_tc_tiling_on_sc=True, needs_layout_passes=False
        ),
        scratch_types=[
            pltpu.VMEM((D, TC + 1), jnp.float32),
            pltpu.VMEM((D, TC + 1), jnp.float32),
            pltpu.VMEM((TC // 2, DP), jnp.float32),
            pltpu.SemaphoreType.DMA,
            pltpu.SemaphoreType.DMA,
        ],
    )
    def repack(wt_hbm, tail_hbm, out_hbm, in0, in1, outv, s0, s1):
        wid = lax.axis_index("s") * 2 + lax.axis_index("c")
        ins = (in0, in1)
        sems = (s0, s1)
        d_idx = [jax.lax.iota(jnp.int32, 16) + k * 16 for k in range(4)]

        def fire(c, buf, sem):
            pltpu.make_async_copy(
                wt_hbm.at[:, pl.ds(c * TC, TC)], buf.at[:, pl.ds(0, TC)], sem
            ).start()

        def drain(buf, sem):
            pltpu.make_async_copy(
                wt_hbm.at[:, pl.ds(0, TC)], buf.at[:, pl.ds(0, TC)], sem
            ).wait()

        fire(wid, in0, s0)

        @pl.when(wid + 32 < NCH)
        def _():
            fire(wid + 32, in1, s1)

        def body(p, carry):
            for b in range(2):
                i = 2 * p + b
                c = wid + i * 32

                @pl.when(c < NCH)
                def _():
                    drain(ins[b], sems[b])

                    def ubody(u, carry2):
                        t0 = jnp.full((16,), 2 * u, jnp.int32)
                        t1 = t0 + 1
                        for k in range(4):
                            outv[u, pl.ds(k * 16, 16)] = plsc.load_gather(
                                ins[b], [d_idx[k], t0]
                            )
                            outv[u, pl.ds(64 + k * 16, 16)] = plsc.load_gather(
                                ins[b], [d_idx[k], t1]
                            )
                        return carry2

                    lax.fori_loop(0, TC // 2, ubody, 0)
                    pltpu.sync_copy(
                        outv, out_hbm.at[pl.ds(c * (TC // 2), TC // 2)]
                    )
                    c2 = c + 64

                    @pl.when(c2 < NCH)
                    def _():
                        fire(c2, ins[b], sems[b])

            return carry

        n_iter = (NCH + 31) // 32
        lax.fori_loop(0, (n_iter + 1) // 2, body, 0)

        @pl.when(wid == 0)
        def _():
            pltpu.sync_copy(tail_hbm, outv.at[pl.ds(0, 32)])
            pltpu.sync_copy(
                outv.at[pl.ds(0, 32)], out_hbm.at[pl.ds(V // 2 - 32, 32)]
            )

    return repack


def _emb_call(total):
    b_per_w = total // NW
    n_rows = b_per_w // CHUNK
    n_bufs = b_per_w // BUF_ROWS

    mesh = plsc.VectorSubcoreMesh(core_axis_name="c", subcore_axis_name="s")

    @functools.partial(
        pl.kernel,
        mesh=mesh,
        out_type=jax.ShapeDtypeStruct((total, DP), jnp.float32),
        compiler_params=pltpu.CompilerParams(use_tc_tiling_on_sc=False),
        scratch_types=[
            pltpu.VMEM((n_rows, CHUNK), jnp.int32),
            pltpu.VMEM((BUF_ROWS, D), jnp.float32),
            pltpu.VMEM((BUF_ROWS, D), jnp.float32),
            pltpu.SemaphoreType.DMA,
            pltpu.SemaphoreType.DMA,
        ],
    )
    def emb(idx_hbm, table_hbm, out_hbm, idx_v, rows0, rows1, g0, g1):
        wid = lax.axis_index("s") * 2 + lax.axis_index("c")
        base = wid * b_per_w
        pltpu.sync_copy(idx_hbm.at[wid], idx_v)

        rows = (rows0, rows1)
        gsem = (g0, g1)

        def fire(g, rows_ref, sem):
            for j in range(STREAMS_PER_BUF):
                pltpu.make_async_copy(
                    table_hbm.at[idx_v.at[g * STREAMS_PER_BUF + j]],
                    rows_ref.at[pl.ds(j * CHUNK, CHUNK)],
                    sem,
                ).start()

        def drain(rows_ref, sem):
            pltpu.make_async_copy(
                table_hbm.at[pl.ds(0, BUF_ROWS)], rows_ref, sem
            ).wait()

        fire(0, rows0, g0)

        def body(p, carry):
            for b in range(2):
                g = p * 2 + b
                drain(rows[b], gsem[b])
                if b == 0:
                    fire(g + 1, rows[1], gsem[1])
                else:
                    @pl.when(g + 1 < n_bufs)
                    def _():
                        fire(g + 1, rows[0], gsem[0])
                pltpu.sync_copy(
                    rows[b],
                    out_hbm.at[pl.ds(base + g * BUF_ROWS, BUF_ROWS), pl.ds(0, D)],
                )
            return carry

        lax.fori_loop(0, n_bufs // 2, body, 0)

    return emb


def kernel(token_ids, weight):
    B, S = token_ids.shape
    total = B * S
    V = weight.shape[0]
    idx = token_ids.reshape(NW, total // (NW * CHUNK), CHUNK).astype(jnp.int32)
    wt = weight.T
    tail = weight[NCH * TC:].reshape(V // 2 - NCH * TC // 2, DP)
    table = _repack_call(V)(wt, tail).reshape(V, D)
    out = _emb_call(total)(idx, table)
    return out[:, :D].reshape(B, S, D)


# speedup vs baseline: 2.0614x; 2.0563x over previous
import functools

import jax
import jax.numpy as jnp
from jax import lax
from jax.experimental import pallas as pl
from jax.experimental.pallas import tpu as pltpu
from jax.experimental.pallas import tpu_sc as plsc

D = 64
DP = 128
NW = 32
CHUNK = 128
STREAMS_PER_BUF = 4
BUF_ROWS = CHUNK * STREAMS_PER_BUF


def _emb_call(total):
    b_per_w = total // NW
    n_rows = b_per_w // CHUNK
    n_bufs = b_per_w // BUF_ROWS

    mesh = plsc.VectorSubcoreMesh(core_axis_name="c", subcore_axis_name="s")

    @functools.partial(
        pl.kernel,
        mesh=mesh,
        out_type=jax.ShapeDtypeStruct((total, DP), jnp.float32),
        compiler_params=pltpu.CompilerParams(use_tc_tiling_on_sc=False),
        scratch_types=[
            pltpu.VMEM((n_rows, CHUNK), jnp.int32),
            pltpu.VMEM((BUF_ROWS, D), jnp.float32),
            pltpu.VMEM((BUF_ROWS, D), jnp.float32),
            pltpu.SemaphoreType.DMA,
            pltpu.SemaphoreType.DMA,
        ],
    )
    def emb(idx_hbm, table_hbm, out_hbm, idx_v, rows0, rows1, g0, g1):
        wid = lax.axis_index("s") * 2 + lax.axis_index("c")
        base = wid * b_per_w
        pltpu.sync_copy(idx_hbm.at[wid], idx_v)

        rows = (rows0, rows1)
        gsem = (g0, g1)

        def fire(g, rows_ref, sem):
            for j in range(STREAMS_PER_BUF):
                pltpu.make_async_copy(
                    table_hbm.at[idx_v.at[g * STREAMS_PER_BUF + j]],
                    rows_ref.at[pl.ds(j * CHUNK, CHUNK)],
                    sem,
                ).start()

        def drain(rows_ref, sem):
            pltpu.make_async_copy(
                table_hbm.at[pl.ds(0, BUF_ROWS)], rows_ref, sem
            ).wait()

        fire(0, rows0, g0)

        def body(p, carry):
            for b in range(2):
                g = p * 2 + b
                drain(rows[b], gsem[b])
                if b == 0:
                    fire(g + 1, rows[1], gsem[1])
                else:
                    @pl.when(g + 1 < n_bufs)
                    def _():
                        fire(g + 1, rows[0], gsem[0])
                pltpu.sync_copy(
                    rows[b],
                    out_hbm.at[pl.ds(base + g * BUF_ROWS, BUF_ROWS), pl.ds(0, D)],
                )
            return carry

        lax.fori_loop(0, n_bufs // 2, body, 0)

    return emb


def kernel(token_ids, weight):
    B, S = token_ids.shape
    total = B * S
    idx = token_ids.reshape(NW, total // (NW * CHUNK), CHUNK).astype(jnp.int32)
    out = _emb_call(total)(idx, weight)
    return out[:, :D].reshape(B, S, D)
